# Initial kernel scaffold; baseline (speedup 1.0000x reference)
import jax
import jax.numpy as jnp
from jax.experimental import pallas as pl


def kernel(x, We, be, W0, b0, W1, b1, Wmu, bmu, Wv, bv, protos):
    raise NotImplementedError("write your pallas kernel here")


# speedup vs baseline: 5.5635x; 5.5635x over previous
import jax
import jax.numpy as jnp
from jax.experimental import pallas as pl
from jax.experimental.pallas import tpu as pltpu

_B, _IN, _H, _C, _K = 512, 768, 64, 256, 1024
_HI = jax.lax.Precision.HIGHEST


def _dot(a, b):
    return jnp.dot(a, b, precision=_HI, preferred_element_type=jnp.float32)


def _vq_body(x_ref, We_ref, be_ref, W0_ref, b0_ref, W1_ref, b1_ref,
             Wmu_ref, bmu_ref, protos_ref, g_ref, out_ref, loss_ref):
    x = x_ref[...]
    emb = _dot(x, We_ref[...]) + be_ref[...]
    h0 = jnp.maximum(_dot(emb, W0_ref[...]) + b0_ref[...], 0.0)
    h1 = jnp.maximum(_dot(h0, W1_ref[...]) + b1_ref[...], 0.0)
    mu = _dot(h1, Wmu_ref[...]) + bmu_ref[...]

    p = protos_ref[...]
    cross = jax.lax.dot_general(mu, p, (((1,), (1,)), ((), ())),
                                precision=_HI, preferred_element_type=jnp.float32)
    mu2 = jnp.sum(mu * mu, axis=1, keepdims=True)
    pp = p * p
    ones_row = jnp.ones((1, _C), jnp.float32)
    p2 = jax.lax.dot_general(ones_row, pp, (((1,), (1,)), ((), ())),
                             precision=_HI, preferred_element_type=jnp.float32)

    y = g_ref[...] + (2.0 * cross - mu2) - p2
    row_max = jnp.max(y, axis=1, keepdims=True)
    shifted = y - row_max
    ey = jnp.exp(shifted)
    sum_ey = jnp.sum(ey, axis=1, keepdims=True)
    logprobs = shifted - jnp.log(sum_ey)
    soft = ey / sum_ey

    idx = jnp.argmax(logprobs, axis=1)
    lanes = jax.lax.broadcasted_iota(jnp.int32, (_B, _K), 1)
    hard = (lanes == idx[:, None]).astype(jnp.float32)
    out_ref[...] = _dot(hard, p)

    prior = jnp.sum(soft, axis=0, keepdims=True) * (1.0 / _B) + 1e-6
    colsum_lp = jnp.sum(logprobs, axis=0, keepdims=True)
    logp = jnp.log(prior)
    capacity = jnp.sum(prior * (_B * logp - colsum_lp), keepdims=True) * (1.0 / _B)
    ent = -jnp.sum(prior * logp, keepdims=True)
    loss_ref[...] = capacity - 0.001 * ent


def kernel(x, We, be, W0, b0, W1, b1, Wmu, bmu, Wv, bv, protos):
    del Wv, bv
    k2 = jax.random.split(jax.random.key(42))[1]
    u = jax.random.uniform(k2, (_B, _K), jnp.float32, 1e-10, 1.0)
    g = -jnp.log(-jnp.log(u))

    out, loss = pl.pallas_call(
        _vq_body,
        out_shape=(
            jax.ShapeDtypeStruct((_B, _C), jnp.float32),
            jax.ShapeDtypeStruct((1, 1), jnp.float32),
        ),
    )(x, We, be.reshape(1, _H), W0, b0.reshape(1, _H), W1, b1.reshape(1, _C),
      Wmu, bmu.reshape(1, _C), protos, g)

    return (out, loss.reshape(()), jnp.zeros(()))
